# baseline (speedup 1.0000x reference)
import functools

import jax
import jax.numpy as jnp
from jax import lax
from jax.experimental import pallas as pl
from jax.experimental.pallas import tpu as pltpu
from jax.experimental.pallas import tpu_sc as plsc

_NUM_CORES = 2
_NUM_SUBCORES = 16
_NUM_WORKERS = _NUM_CORES * _NUM_SUBCORES

_CHUNK = 32
_NBUF = 3


@functools.lru_cache(maxsize=None)
def _make_embed(n_tokens: int, vocab: int, d_model: int):
    assert n_tokens % _NUM_WORKERS == 0
    per_worker = n_tokens // _NUM_WORKERS
    assert per_worker % _CHUNK == 0
    n_chunks = per_worker // _CHUNK

    mesh = plsc.VectorSubcoreMesh(
        core_axis_name="c", subcore_axis_name="s",
        num_cores=_NUM_CORES, num_subcores=_NUM_SUBCORES)

    @functools.partial(
        pl.kernel,
        mesh=mesh,
        out_type=jax.ShapeDtypeStruct((n_tokens, d_model), jnp.float32),
        scratch_types=[
            pltpu.VMEM((per_worker,), jnp.int32),
            [pltpu.VMEM((_CHUNK, d_model), jnp.float32) for _ in range(_NBUF)],
            [pltpu.SemaphoreType.DMA for _ in range(_NBUF)],
            [pltpu.SemaphoreType.DMA for _ in range(_NBUF)],
        ],
    )
    def embed(table_hbm, idx_hbm, out_hbm, idx_v, bufs, gsems, wsems):
        wid = lax.axis_index("s") * _NUM_CORES + lax.axis_index("c")
        base = wid * per_worker
        pltpu.sync_copy(idx_hbm.at[pl.ds(base, per_worker)], idx_v)

        def gather(c, s):
            return pltpu.async_copy(
                table_hbm.at[idx_v.at[pl.ds(c * _CHUNK, _CHUNK)]],
                bufs[s], gsems[s])

        def write(c, s):
            return pltpu.async_copy(
                bufs[s], out_hbm.at[pl.ds(base + c * _CHUNK, _CHUNK)],
                wsems[s])

        gcp = [gather(s, s) for s in range(_NBUF)]
        wcp = [None] * _NBUF
        for c in range(n_chunks):
            s = c % _NBUF
            gcp[s].wait()
            wcp[s] = write(c, s)
            if c >= 1 and c - 1 + _NBUF < n_chunks:
                ps = (c - 1) % _NBUF
                wcp[ps].wait()
                gcp[ps] = gather(c - 1 + _NBUF, ps)
        for c in range(max(0, n_chunks - _NBUF), n_chunks):
            wcp[c % _NBUF].wait()

    return embed


def kernel(tokens, W_E):
    batch, seq = tokens.shape
    vocab, d_model = W_E.shape
    idx = tokens.reshape(batch * seq).astype(jnp.int32)
    embed = _make_embed(batch * seq, vocab, d_model)
    out = embed(W_E, idx)
    return out.reshape(batch, seq, d_model)


# speedup vs baseline: 1.0233x; 1.0233x over previous
import functools

import jax
import jax.numpy as jnp
from jax import lax
from jax.experimental import pallas as pl
from jax.experimental.pallas import tpu as pltpu
from jax.experimental.pallas import tpu_sc as plsc

_NUM_CORES = 2
_NUM_SUBCORES = 16
_NUM_WORKERS = _NUM_CORES * _NUM_SUBCORES

_CHUNK = 16
_NBUF = 6


@functools.lru_cache(maxsize=None)
def _make_embed(n_tokens: int, vocab: int, d_model: int):
    assert n_tokens % _NUM_WORKERS == 0
    per_worker = n_tokens // _NUM_WORKERS
    assert per_worker % _CHUNK == 0
    n_chunks = per_worker // _CHUNK

    mesh = plsc.VectorSubcoreMesh(
        core_axis_name="c", subcore_axis_name="s",
        num_cores=_NUM_CORES, num_subcores=_NUM_SUBCORES)

    @functools.partial(
        pl.kernel,
        mesh=mesh,
        out_type=jax.ShapeDtypeStruct((n_tokens, d_model), jnp.float32),
        scratch_types=[
            pltpu.VMEM((per_worker,), jnp.int32),
            [pltpu.VMEM((_CHUNK, d_model), jnp.float32) for _ in range(_NBUF)],
            [pltpu.SemaphoreType.DMA for _ in range(_NBUF)],
            [pltpu.SemaphoreType.DMA for _ in range(_NBUF)],
        ],
    )
    def embed(table_hbm, idx_hbm, out_hbm, idx_v, bufs, gsems, wsems):
        wid = lax.axis_index("s") * _NUM_CORES + lax.axis_index("c")
        base = wid * per_worker
        pltpu.sync_copy(idx_hbm.at[pl.ds(base, per_worker)], idx_v)

        def gather(c, s):
            return pltpu.async_copy(
                table_hbm.at[idx_v.at[pl.ds(c * _CHUNK, _CHUNK)]],
                bufs[s], gsems[s])

        def write(c, s):
            return pltpu.async_copy(
                bufs[s], out_hbm.at[pl.ds(base + c * _CHUNK, _CHUNK)],
                wsems[s])

        gcp = [gather(s, s) for s in range(_NBUF)]
        wcp = [None] * _NBUF
        for c in range(n_chunks):
            s = c % _NBUF
            gcp[s].wait()
            wcp[s] = write(c, s)
            if c >= 1 and c - 1 + _NBUF < n_chunks:
                ps = (c - 1) % _NBUF
                wcp[ps].wait()
                gcp[ps] = gather(c - 1 + _NBUF, ps)
        for c in range(max(0, n_chunks - _NBUF), n_chunks):
            wcp[c % _NBUF].wait()

    return embed


def kernel(tokens, W_E):
    batch, seq = tokens.shape
    vocab, d_model = W_E.shape
    idx = tokens.reshape(batch * seq).astype(jnp.int32)
    embed = _make_embed(batch * seq, vocab, d_model)
    out = embed(W_E, idx)
    return out.reshape(batch, seq, d_model)
